# Initial kernel scaffold; baseline (speedup 1.0000x reference)
import jax
import jax.numpy as jnp
from jax.experimental import pallas as pl


def kernel(input_ids, params):
    raise NotImplementedError("write your pallas kernel here")


# speedup vs baseline: 36.7432x; 36.7432x over previous
import functools

import jax
import jax.numpy as jnp
import numpy as np
from jax import lax
from jax.experimental import pallas as pl
from jax.experimental.pallas import tpu as pltpu
from jax.experimental.pallas import tpu_sc as plsc

B, S, H, V, HCM, SL, L, I = 2, 2048, 768, 8192, 256, 32, 2, 3072

TS = 512
LT_R = 1024
LT_V = 2048


def _decay_weights():
    t = np.arange(S)
    n_flags = ((S - 1) // 10) + 1
    m = np.ceil(t / 10).astype(np.int64)
    pow9 = np.ones(n_flags + 1, dtype=np.float64)
    for k in range(1, n_flags + 1):
        pow9[k] = pow9[k - 1] * 0.9
    w = pow9[n_flags - m].astype(np.float32)
    return jnp.asarray(w.reshape(S // TS, 1, TS))


def _sc_embed_gather(table, idx_flat):
    info = plsc.get_sparse_core_info()
    nc, ns = info.num_cores, info.num_subcores
    nw = nc * ns
    rows = idx_flat.shape[0]
    bpw = rows // nw
    mesh = plsc.VectorSubcoreMesh(core_axis_name="c", subcore_axis_name="s")

    @functools.partial(
        pl.kernel,
        mesh=mesh,
        out_type=jax.ShapeDtypeStruct((rows, H), jnp.float32),
        scratch_types=[
            pltpu.VMEM((bpw,), jnp.int32),
            pltpu.VMEM((bpw, H), jnp.float32),
            pltpu.SemaphoreType.DMA,
        ],
    )
    def gather_k(table_hbm, idx_hbm, out_hbm, idx_v, rows_v, sem):
        wid = lax.axis_index("s") * nc + lax.axis_index("c")
        base = wid * bpw
        pltpu.sync_copy(idx_hbm.at[pl.ds(base, bpw)], idx_v)
        pltpu.async_copy(table_hbm.at[idx_v], rows_v, sem).wait()
        pltpu.sync_copy(rows_v, out_hbm.at[pl.ds(base, bpw)])

    return gather_k(table, idx_flat)


def _ln(x, g, b):
    m = x.mean(-1, keepdims=True)
    v = ((x - m) ** 2).mean(-1, keepdims=True)
    return (x - m) / jnp.sqrt(v + 1e-5) * g + b


def _dot(a, b):
    return jnp.dot(a, b, preferred_element_type=jnp.float32)


def _make_layer_body(add_pos, final_ln):
    def body(*refs):
        it = iter(refs)
        h_ref = next(it)
        pos_ref = next(it) if add_pos else None
        w_ref = next(it)
        wi, bi = next(it), next(it)
        wg, bg = next(it), next(it)
        wq, bq = next(it), next(it)
        wo, bo = next(it), next(it)
        ln_g, ln_b = next(it), next(it)
        f1, fb1 = next(it), next(it)
        f2, fb2 = next(it), next(it)
        fln_g, fln_b = next(it), next(it)
        if final_ln:
            fin_g, fin_b = next(it), next(it)
        h_out = next(it)
        fast_out = next(it)
        slow_out = next(it)
        hln_out = next(it) if final_ln else None

        t = pl.program_id(1)
        h = h_ref[0]
        if add_pos:
            h = h + pos_ref[...]

        items = _dot(h, wi[...]) + bi[...]
        fg = jax.nn.sigmoid(_dot(h, wg[...]) + bg[...])
        wv = w_ref[0]
        wfg = fg * wv.reshape(TS, 1)
        dn = (((0,), (0,)), ((), ()))
        fa = lax.dot_general(wfg, items, dn,
                             preferred_element_type=jnp.float32)
        sa = lax.dot_general(fg - wfg, items, dn,
                             preferred_element_type=jnp.float32)

        @pl.when(t == 0)
        def _():
            fast_out[0] = fa
            slow_out[0] = sa

        @pl.when(t != 0)
        def _():
            fast_out[0] += fa
            slow_out[0] += sa

        query = _dot(h, wq[...]) + bq[...]
        out = _dot(query, wo[...]) + bo[...]
        h1 = _ln(h + out, ln_g[...], ln_b[...])
        ffn = _dot(jax.nn.gelu(_dot(h1, f1[...]) + fb1[...]), f2[...]) + fb2[...]
        h2 = _ln(h1 + ffn, fln_g[...], fln_b[...])
        h_out[0] = h2
        if final_ln:
            hln_out[0] = _ln(h2, fin_g[...], fin_b[...])

    return body


def _layer_call(h, p, w3, pos=None, final=None):
    add_pos = pos is not None
    final_ln = final is not None
    row = lambda x: x.reshape(1, -1)
    full2 = lambda a: pl.BlockSpec(a.shape, lambda b, t: (0, 0))

    inputs = [h]
    in_specs = [pl.BlockSpec((1, TS, H), lambda b, t: (b, t, 0))]
    if add_pos:
        inputs.append(pos)
        in_specs.append(pl.BlockSpec((TS, H), lambda b, t: (t, 0)))
    inputs.append(w3)
    in_specs.append(pl.BlockSpec((1, 1, TS), lambda b, t: (t, 0, 0)))

    wmats = [
        p['Wi'], row(p['bi']),
        p['Wg'][:H, :SL], row(p['bg'][:SL]),
        p['Wq'], row(p['bq']),
        p['Wo'][:H], row(p['bo']),
        row(p['ln_g']), row(p['ln_b']),
        p['F1'], row(p['fb1']),
        p['F2'], row(p['fb2']),
        row(p['fln_g']), row(p['fln_b']),
    ]
    if final_ln:
        wmats += [row(final[0]), row(final[1])]
    inputs += wmats
    in_specs += [full2(a) for a in wmats]

    out_shape = [
        jax.ShapeDtypeStruct((B, S, H), jnp.float32),
        jax.ShapeDtypeStruct((B, SL, HCM), jnp.float32),
        jax.ShapeDtypeStruct((B, SL, HCM), jnp.float32),
    ]
    out_specs = [
        pl.BlockSpec((1, TS, H), lambda b, t: (b, t, 0)),
        pl.BlockSpec((1, SL, HCM), lambda b, t: (b, 0, 0)),
        pl.BlockSpec((1, SL, HCM), lambda b, t: (b, 0, 0)),
    ]
    if final_ln:
        out_shape.append(jax.ShapeDtypeStruct((B, S, H), jnp.float32))
        out_specs.append(pl.BlockSpec((1, TS, H), lambda b, t: (b, t, 0)))

    return pl.pallas_call(
        _make_layer_body(add_pos, final_ln),
        grid=(B, S // TS),
        in_specs=in_specs,
        out_specs=out_specs,
        out_shape=out_shape,
    )(*inputs)


def _logits_body(h_ref, emb_ref, out_ref):
    out_ref[...] = lax.dot_general(
        h_ref[...], emb_ref[...], (((1,), (1,)), ((), ())),
        preferred_element_type=jnp.float32)


def _logits_call(hln_flat, tok_emb):
    rows = hln_flat.shape[0]
    return pl.pallas_call(
        _logits_body,
        grid=(V // LT_V, rows // LT_R),
        in_specs=[
            pl.BlockSpec((LT_R, H), lambda v, r: (r, 0)),
            pl.BlockSpec((LT_V, H), lambda v, r: (v, 0)),
        ],
        out_specs=pl.BlockSpec((LT_R, LT_V), lambda v, r: (r, v)),
        out_shape=jax.ShapeDtypeStruct((rows, V), jnp.float32),
    )(hln_flat, tok_emb)


def kernel(input_ids, params):
    ids_flat = input_ids.reshape(-1).astype(jnp.int32)
    emb = _sc_embed_gather(params['tok_emb'], ids_flat)
    h = emb.reshape(B, S, H)
    w3 = _decay_weights()
    pos = params['pos_emb'][:S]

    layers = params['layers']
    fasts, slows = [], []
    for li, p in enumerate(layers):
        pos_arg = pos if li == 0 else None
        final = (params['final_g'], params['final_b']) if li == L - 1 else None
        res = _layer_call(h, p, w3, pos=pos_arg, final=final)
        if final is not None:
            h, fast, slow, hln = res
        else:
            h, fast, slow = res
        fasts.append(fast)
        slows.append(slow)

    logits = _logits_call(hln.reshape(B * S, H), params['tok_emb'])
    return logits.reshape(B, S, V), jnp.stack(fasts), jnp.stack(slows)
